# Initial kernel scaffold; baseline (speedup 1.0000x reference)
import functools

import jax
import jax.numpy as jnp
from jax import lax
from jax.experimental import pallas as pl
from jax.experimental.pallas import tpu as pltpu
from jax.experimental.pallas import tpu_sc as plsc

N = 16384
K = 50
V1, D = 1000, 2
V2 = 100
NC, NS, L = 2, 16, 16
NW = NC * NS
RPW = N // NW
G = RPW // L

_mesh = plsc.VectorSubcoreMesh(core_axis_name="c", subcore_axis_name="s")


@functools.partial(
    pl.kernel,
    out_type=[
        jax.ShapeDtypeStruct((N * D,), jnp.float32),
        jax.ShapeDtypeStruct((N * D,), jnp.float32),
    ],
    mesh=_mesh,
    scratch_types=[
        pltpu.VMEM((RPW,), jnp.int32),
        pltpu.VMEM((RPW * K,), jnp.int32),
        pltpu.VMEM((V1 * D,), jnp.float32),
        pltpu.VMEM((V2 * D,), jnp.float32),
        pltpu.VMEM((RPW * D,), jnp.float32),
        pltpu.VMEM((RPW * D,), jnp.float32),
    ],
)
def _sc_kernel(idx_hbm, bag_hbm, w1_hbm, w2_hbm, out1_hbm, out2_hbm,
               idx_v, bag_v, w1_v, w2_v, o1_v, o2_v):
    wid = lax.axis_index("s") * NC + lax.axis_index("c")
    base = wid * RPW

    pltpu.sync_copy(idx_hbm.at[pl.ds(base, RPW)], idx_v)
    pltpu.sync_copy(bag_hbm.at[pl.ds(base * K, RPW * K)], bag_v)
    pltpu.sync_copy(w1_hbm, w1_v)
    pltpu.sync_copy(w2_hbm, w2_v)

    lanes = lax.iota(jnp.int32, L)
    lanes_k = lanes * K
    lanes_2 = lanes * 2

    def o1_body(j, carry):
        idx = idx_v[pl.ds(j * L, L)]
        i2 = idx * 2
        c0 = jnp.maximum(plsc.load_gather(w1_v, [i2]), 0.0)
        c1 = jnp.maximum(plsc.load_gather(w1_v, [i2 + 1]), 0.0)
        pos = lanes_2 + j * (2 * L)
        plsc.store_scatter(o1_v, [pos], c0)
        plsc.store_scatter(o1_v, [pos + 1], c1)
        return carry

    lax.fori_loop(0, G, o1_body, 0)

    def o2_body(j, carry):
        gbase = j * (L * K)

        def bag_step(k, accs):
            a0, a1 = accs
            b = plsc.load_gather(bag_v, [lanes_k + (gbase + k)])
            b2 = b * 2
            a0 = a0 + plsc.load_gather(w2_v, [b2])
            a1 = a1 + plsc.load_gather(w2_v, [b2 + 1])
            return (a0, a1)

        zero = jnp.zeros((L,), jnp.float32)
        a0, a1 = lax.fori_loop(0, K, bag_step, (zero, zero))
        pos = lanes_2 + j * (2 * L)
        plsc.store_scatter(o2_v, [pos], a0)
        plsc.store_scatter(o2_v, [pos + 1], a1)
        return carry

    lax.fori_loop(0, G, o2_body, 0)

    pltpu.sync_copy(o1_v, out1_hbm.at[pl.ds(base * D, RPW * D)])
    pltpu.sync_copy(o2_v, out2_hbm.at[pl.ds(base * D, RPW * D)])


def kernel(indices, bag_indices, W1, W2):
    out1, out2 = _sc_kernel(
        indices, bag_indices.reshape(-1), W1.reshape(-1), W2.reshape(-1)
    )
    return out1.reshape(N, D), out2.reshape(N, D)


# speedup vs baseline: 35.3921x; 35.3921x over previous
import functools

import jax
import jax.numpy as jnp
from jax import lax
from jax.experimental import pallas as pl
from jax.experimental.pallas import tpu as pltpu
from jax.experimental.pallas import tpu_sc as plsc

N = 16384
K = 50
V1, D = 1000, 2
V2 = 100
NC, NS, L = 2, 16, 16
NW = NC * NS
RPW = N // NW
G = RPW // L

_mesh = plsc.VectorSubcoreMesh(core_axis_name="c", subcore_axis_name="s")


@functools.partial(
    pl.kernel,
    out_type=[
        jax.ShapeDtypeStruct((N * D,), jnp.float32),
        jax.ShapeDtypeStruct((N * D,), jnp.float32),
    ],
    mesh=_mesh,
    compiler_params=pltpu.CompilerParams(needs_layout_passes=False),
    scratch_types=[
        pltpu.VMEM((RPW,), jnp.int32),
        pltpu.VMEM((RPW * K,), jnp.int32),
        pltpu.VMEM((V1 * D,), jnp.float32),
        pltpu.VMEM((V2 * D,), jnp.float32),
        pltpu.VMEM((RPW * D,), jnp.float32),
        pltpu.VMEM((RPW * D,), jnp.float32),
    ],
)
def _sc_kernel(idx_hbm, bag_hbm, w1_hbm, w2_hbm, out1_hbm, out2_hbm,
               idx_v, bag_v, w1_v, w2_v, o1_v, o2_v):
    wid = lax.axis_index("s") * NC + lax.axis_index("c")
    base = wid * RPW

    pltpu.sync_copy(idx_hbm.at[pl.ds(base, RPW)], idx_v)
    pltpu.sync_copy(bag_hbm.at[pl.ds(base * K, RPW * K)], bag_v)
    pltpu.sync_copy(w1_hbm, w1_v)
    pltpu.sync_copy(w2_hbm, w2_v)

    lanes = lax.iota(jnp.int32, L)
    lanes_k = lanes * K
    lanes_2 = lanes * 2

    def o1_body(j, carry):
        idx = idx_v[pl.ds(j * L, L)]
        i2 = idx * 2
        c0 = jnp.maximum(plsc.load_gather(w1_v, [i2]), 0.0)
        c1 = jnp.maximum(plsc.load_gather(w1_v, [i2 + 1]), 0.0)
        pos = lanes_2 + j * (2 * L)
        plsc.store_scatter(o1_v, [pos], c0)
        plsc.store_scatter(o1_v, [pos + 1], c1)
        return carry

    lax.fori_loop(0, G, o1_body, 0)

    def o2_body(j, carry):
        gbase = j * (L * K)

        def bag_step(k, accs):
            a0, a1 = accs
            b = plsc.load_gather(bag_v, [lanes_k + (gbase + k)])
            b2 = b * 2
            a0 = a0 + plsc.load_gather(w2_v, [b2])
            a1 = a1 + plsc.load_gather(w2_v, [b2 + 1])
            return (a0, a1)

        zero = jnp.zeros((L,), jnp.float32)
        a0, a1 = lax.fori_loop(0, K, bag_step, (zero, zero))
        pos = lanes_2 + j * (2 * L)
        plsc.store_scatter(o2_v, [pos], a0)
        plsc.store_scatter(o2_v, [pos + 1], a1)
        return carry

    lax.fori_loop(0, G, o2_body, 0)

    pltpu.sync_copy(o1_v, out1_hbm.at[pl.ds(base * D, RPW * D)])
    pltpu.sync_copy(o2_v, out2_hbm.at[pl.ds(base * D, RPW * D)])


def kernel(indices, bag_indices, W1, W2):
    out1, out2 = _sc_kernel(
        indices, bag_indices.reshape(-1), W1.reshape(-1), W2.reshape(-1)
    )
    return out1.reshape(N, D), out2.reshape(N, D)
